# Initial kernel scaffold; baseline (speedup 1.0000x reference)
import jax
import jax.numpy as jnp
from jax.experimental import pallas as pl


def kernel(image, feat, W1, b1, Wc, bc, Wr, br):
    raise NotImplementedError("write your pallas kernel here")


# speedup vs baseline: 1.0053x; 1.0053x over previous
import math
import functools

import jax
import jax.numpy as jnp
from jax import lax
from jax.experimental import pallas as pl
import numpy as np

_SCALES = [128.0, 256.0, 512.0]
_ASPECT_RATIOS = [0.5, 1.0, 2.0]
_NUM_ANCHORS = 9
_PRENMS_TOPK = 12000
_NMS_THRESH = 0.7
_RPN_TOPK = 2000
_MIN_SIZE = 16.0
_BBOX_CLIP = math.log(1000.0 / 16.0)
_C_IN = 384


def _conv_kernel(x9_ref, w1_ref, b1_ref, wc_ref, bc_ref, wr_ref, br_ref,
                 cls_ref, reg_ref):
    x9 = x9_ref[...]
    y = jax.lax.dot_general(
        w1_ref[...], x9, (((1,), (0,)), ((), ())),
        preferred_element_type=jnp.float32)
    y = jnp.maximum(y + b1_ref[...][:, None], 0.0)
    cls = jax.lax.dot_general(
        wc_ref[...], y, (((1,), (0,)), ((), ())),
        preferred_element_type=jnp.float32)
    cls_ref[...] = cls + bc_ref[...][:, None]
    reg = jax.lax.dot_general(
        wr_ref[...], y, (((1,), (0,)), ((), ())),
        preferred_element_type=jnp.float32)
    reg_ref[...] = reg + br_ref[...][:, None]


def _conv_backbone(feat, W1, b1, Wc, bc, Wr, br):
    fp = jnp.pad(feat[0], ((0, 0), (1, 1), (1, 1)))
    shifts = [fp[:, ky:ky + 64, kx:kx + 64].reshape(_C_IN, 4096)
              for ky in range(3) for kx in range(3)]
    x9 = jnp.concatenate(shifts, axis=0)
    w1r = jnp.transpose(W1, (0, 2, 3, 1)).reshape(_C_IN, 9 * _C_IN)
    wcr = Wc.reshape(_NUM_ANCHORS, _C_IN)
    wrr = Wr.reshape(_NUM_ANCHORS * 4, _C_IN)

    nblk = 4
    bs = 4096 // nblk
    cls, reg = pl.pallas_call(
        _conv_kernel,
        grid=(nblk,),
        in_specs=[
            pl.BlockSpec((9 * _C_IN, bs), lambda j: (0, j)),
            pl.BlockSpec((_C_IN, 9 * _C_IN), lambda j: (0, 0)),
            pl.BlockSpec((_C_IN,), lambda j: (0,)),
            pl.BlockSpec((_NUM_ANCHORS, _C_IN), lambda j: (0, 0)),
            pl.BlockSpec((_NUM_ANCHORS,), lambda j: (0,)),
            pl.BlockSpec((_NUM_ANCHORS * 4, _C_IN), lambda j: (0, 0)),
            pl.BlockSpec((_NUM_ANCHORS * 4,), lambda j: (0,)),
        ],
        out_specs=[
            pl.BlockSpec((_NUM_ANCHORS, bs), lambda j: (0, j)),
            pl.BlockSpec((_NUM_ANCHORS * 4, bs), lambda j: (0, j)),
        ],
        out_shape=[
            jax.ShapeDtypeStruct((_NUM_ANCHORS, 4096), jnp.float32),
            jax.ShapeDtypeStruct((_NUM_ANCHORS * 4, 4096), jnp.float32),
        ],
    )(x9, w1r, b1, wcr, bc, wrr, br)
    return cls, reg


def _gen_anchors(image_h, image_w, grid_h, grid_w, dtype):
    stride_h = image_h // grid_h
    stride_w = image_w // grid_w
    scales = jnp.asarray(_SCALES, dtype)
    ar = jnp.asarray(_ASPECT_RATIOS, dtype)
    h_ratios = jnp.sqrt(ar)
    w_ratios = 1.0 / h_ratios
    ws = (w_ratios[:, None] * scales[None, :]).reshape(-1)
    hs = (h_ratios[:, None] * scales[None, :]).reshape(-1)
    base = jnp.round(jnp.stack([-ws, -hs, ws, hs], axis=1) / 2.0)
    sx = jnp.arange(grid_w, dtype=dtype) * stride_w
    sy = jnp.arange(grid_h, dtype=dtype) * stride_h
    syy, sxx = jnp.meshgrid(sy, sx, indexing="ij")
    sxx = sxx.reshape(-1)
    syy = syy.reshape(-1)
    shifts = jnp.stack([sxx, syy, sxx, syy], axis=1)
    return (shifts[:, None, :] + base[None, :, :]).reshape(-1, 4)


def _decode(anchors, deltas):
    w = anchors[:, 2] - anchors[:, 0]
    h = anchors[:, 3] - anchors[:, 1]
    cx = anchors[:, 0] + 0.5 * w
    cy = anchors[:, 1] + 0.5 * h
    dx, dy = deltas[:, 0], deltas[:, 1]
    dw = jnp.clip(deltas[:, 2], -_BBOX_CLIP, _BBOX_CLIP)
    dh = jnp.clip(deltas[:, 3], -_BBOX_CLIP, _BBOX_CLIP)
    pcx = dx * w + cx
    pcy = dy * h + cy
    pw = jnp.exp(dw) * w
    ph = jnp.exp(dh) * h
    return jnp.stack([pcx - 0.5 * pw, pcy - 0.5 * ph,
                      pcx + 0.5 * pw, pcy + 0.5 * ph], axis=1)


def _nms_mask(boxes, valid, thresh):
    n = boxes.shape[0]
    x1, y1, x2, y2 = boxes[:, 0], boxes[:, 1], boxes[:, 2], boxes[:, 3]
    areas = (x2 - x1) * (y2 - y1)
    order = jnp.arange(n)

    def body(i, keep):
        xx1 = jnp.maximum(x1[i], x1)
        yy1 = jnp.maximum(y1[i], y1)
        xx2 = jnp.minimum(x2[i], x2)
        yy2 = jnp.minimum(y2[i], y2)
        inter = jnp.clip(xx2 - xx1, 0.0) * jnp.clip(yy2 - yy1, 0.0)
        iou = inter / (areas[i] + areas - inter + 1e-9)
        suppress = (iou > thresh) & (order > i) & keep[i]
        return keep & (~suppress)

    return lax.fori_loop(0, n, body, valid)


def kernel(image, feat, W1, b1, Wc, bc, Wr, br):
    N, _, Hf, Wf = feat.shape
    H_img, W_img = image.shape[-2], image.shape[-1]
    cls_mat, reg_mat = _conv_backbone(feat, W1, b1, Wc, bc, Wr, br)
    cls_scores = cls_mat.T.reshape(-1)
    reg_flat = reg_mat.reshape(_NUM_ANCHORS, 4, 4096).transpose(2, 0, 1).reshape(-1, 4)
    anchors = _gen_anchors(H_img, W_img, Hf, Wf, feat.dtype)
    proposals = _decode(anchors, lax.stop_gradient(reg_flat))
    scores = jax.nn.sigmoid(cls_scores)
    k = min(_PRENMS_TOPK, scores.shape[0])
    top_scores, top_idx = lax.top_k(scores, k)
    props = proposals[top_idx]
    props = jnp.stack([
        jnp.clip(props[:, 0], 0.0, W_img),
        jnp.clip(props[:, 1], 0.0, H_img),
        jnp.clip(props[:, 2], 0.0, W_img),
        jnp.clip(props[:, 3], 0.0, H_img),
    ], axis=1)
    ws = props[:, 2] - props[:, 0]
    hs = props[:, 3] - props[:, 1]
    valid = (ws >= _MIN_SIZE) & (hs >= _MIN_SIZE)
    top_scores = jnp.where(valid, top_scores, -1.0)
    keep = _nms_mask(props, valid, _NMS_THRESH)
    final_all = jnp.where(keep, top_scores, -1.0)
    final_scores, final_idx = lax.top_k(final_all, _RPN_TOPK)
    final_boxes = props[final_idx]
    return (final_boxes, final_scores)


# speedup vs baseline: 37.1749x; 36.9793x over previous
import math
import functools

import jax
import jax.numpy as jnp
from jax import lax
from jax.experimental import pallas as pl
from jax.experimental.pallas import tpu as pltpu
import numpy as np

_SCALES = [128.0, 256.0, 512.0]
_ASPECT_RATIOS = [0.5, 1.0, 2.0]
_NUM_ANCHORS = 9
_PRENMS_TOPK = 12000
_NMS_THRESH = 0.7
_RPN_TOPK = 2000
_MIN_SIZE = 16.0
_BBOX_CLIP = math.log(1000.0 / 16.0)
_C_IN = 384


def _conv_kernel(x9_ref, w1_ref, b1_ref, wc_ref, bc_ref, wr_ref, br_ref,
                 cls_ref, reg_ref):
    x9 = x9_ref[...]
    y = jax.lax.dot_general(
        w1_ref[...], x9, (((1,), (0,)), ((), ())),
        preferred_element_type=jnp.float32)
    y = jnp.maximum(y + b1_ref[...][:, None], 0.0)
    cls = jax.lax.dot_general(
        wc_ref[...], y, (((1,), (0,)), ((), ())),
        preferred_element_type=jnp.float32)
    cls_ref[...] = cls + bc_ref[...][:, None]
    reg = jax.lax.dot_general(
        wr_ref[...], y, (((1,), (0,)), ((), ())),
        preferred_element_type=jnp.float32)
    reg_ref[...] = reg + br_ref[...][:, None]


def _conv_backbone(feat, W1, b1, Wc, bc, Wr, br):
    fp = jnp.pad(feat[0], ((0, 0), (1, 1), (1, 1)))
    shifts = [fp[:, ky:ky + 64, kx:kx + 64].reshape(_C_IN, 4096)
              for ky in range(3) for kx in range(3)]
    x9 = jnp.concatenate(shifts, axis=0)
    w1r = jnp.transpose(W1, (0, 2, 3, 1)).reshape(_C_IN, 9 * _C_IN)
    wcr = Wc.reshape(_NUM_ANCHORS, _C_IN)
    wrr = Wr.reshape(_NUM_ANCHORS * 4, _C_IN)

    nblk = 4
    bs = 4096 // nblk
    cls, reg = pl.pallas_call(
        _conv_kernel,
        grid=(nblk,),
        in_specs=[
            pl.BlockSpec((9 * _C_IN, bs), lambda j: (0, j)),
            pl.BlockSpec((_C_IN, 9 * _C_IN), lambda j: (0, 0)),
            pl.BlockSpec((_C_IN,), lambda j: (0,)),
            pl.BlockSpec((_NUM_ANCHORS, _C_IN), lambda j: (0, 0)),
            pl.BlockSpec((_NUM_ANCHORS,), lambda j: (0,)),
            pl.BlockSpec((_NUM_ANCHORS * 4, _C_IN), lambda j: (0, 0)),
            pl.BlockSpec((_NUM_ANCHORS * 4,), lambda j: (0,)),
        ],
        out_specs=[
            pl.BlockSpec((_NUM_ANCHORS, bs), lambda j: (0, j)),
            pl.BlockSpec((_NUM_ANCHORS * 4, bs), lambda j: (0, j)),
        ],
        out_shape=[
            jax.ShapeDtypeStruct((_NUM_ANCHORS, 4096), jnp.float32),
            jax.ShapeDtypeStruct((_NUM_ANCHORS * 4, 4096), jnp.float32),
        ],
    )(x9, w1r, b1, wcr, bc, wrr, br)
    return cls, reg


def _gen_anchors(image_h, image_w, grid_h, grid_w, dtype):
    stride_h = image_h // grid_h
    stride_w = image_w // grid_w
    scales = jnp.asarray(_SCALES, dtype)
    ar = jnp.asarray(_ASPECT_RATIOS, dtype)
    h_ratios = jnp.sqrt(ar)
    w_ratios = 1.0 / h_ratios
    ws = (w_ratios[:, None] * scales[None, :]).reshape(-1)
    hs = (h_ratios[:, None] * scales[None, :]).reshape(-1)
    base = jnp.round(jnp.stack([-ws, -hs, ws, hs], axis=1) / 2.0)
    sx = jnp.arange(grid_w, dtype=dtype) * stride_w
    sy = jnp.arange(grid_h, dtype=dtype) * stride_h
    syy, sxx = jnp.meshgrid(sy, sx, indexing="ij")
    sxx = sxx.reshape(-1)
    syy = syy.reshape(-1)
    shifts = jnp.stack([sxx, syy, sxx, syy], axis=1)
    return (shifts[:, None, :] + base[None, :, :]).reshape(-1, 4)


def _decode(anchors, deltas):
    w = anchors[:, 2] - anchors[:, 0]
    h = anchors[:, 3] - anchors[:, 1]
    cx = anchors[:, 0] + 0.5 * w
    cy = anchors[:, 1] + 0.5 * h
    dx, dy = deltas[:, 0], deltas[:, 1]
    dw = jnp.clip(deltas[:, 2], -_BBOX_CLIP, _BBOX_CLIP)
    dh = jnp.clip(deltas[:, 3], -_BBOX_CLIP, _BBOX_CLIP)
    pcx = dx * w + cx
    pcy = dy * h + cy
    pw = jnp.exp(dw) * w
    ph = jnp.exp(dh) * h
    return jnp.stack([pcx - 0.5 * pw, pcy - 0.5 * ph,
                      pcx + 0.5 * pw, pcy + 0.5 * ph], axis=1)


_ROWS = 96
_LANES = 128
_NPAD = _ROWS * _LANES


def _iou_gt(bx1, by1, bx2, by2, ba, rx1, ry1, rx2, ry2, ra):
    xx1 = jnp.maximum(bx1, rx1)
    yy1 = jnp.maximum(by1, ry1)
    xx2 = jnp.minimum(bx2, rx2)
    yy2 = jnp.minimum(by2, ry2)
    inter = jnp.clip(xx2 - xx1, 0.0) * jnp.clip(yy2 - yy1, 0.0)
    iou = inter / (ba + ra - inter + 1e-9)
    return iou > _NMS_THRESH


def _dotf(a, b, dims):
    return jax.lax.dot_general(a, b, (dims, ((), ())),
                               precision=jax.lax.Precision.HIGHEST,
                               preferred_element_type=jnp.float32)


def _nms_kernel(sc_ref, x1_ref, y1_ref, x2_ref, y2_ref, out_ref, keep_ref,
                x1s, y1s, x2s, y2s, ars, rank_s, fsc_s):
    f32 = jnp.float32
    sc = sc_ref[...]
    x1 = jnp.clip(x1_ref[...], 0.0, 512.0)
    y1 = jnp.clip(y1_ref[...], 0.0, 512.0)
    x2 = jnp.clip(x2_ref[...], 0.0, 512.0)
    y2 = jnp.clip(y2_ref[...], 0.0, 512.0)
    rown = jax.lax.broadcasted_iota(jnp.int32, (_ROWS, _LANES), 0)
    lanen = jax.lax.broadcasted_iota(jnp.int32, (_ROWS, _LANES), 1)
    nglob = rown * _LANES + lanen
    real = nglob < 12000
    ws = x2 - x1
    hs = y2 - y1
    valid = (ws >= _MIN_SIZE) & (hs >= _MIN_SIZE) & real
    areas = ws * hs
    keep_ref[...] = valid.astype(f32)
    x1s[...] = x1
    y1s[...] = y1
    x2s[...] = x2
    y2s[...] = y2
    ars[...] = areas

    li = jax.lax.broadcasted_iota(jnp.int32, (_LANES, _LANES), 0)
    lj = jax.lax.broadcasted_iota(jnp.int32, (_LANES, _LANES), 1)
    tri = (li < lj).astype(f32)
    row_iota = jax.lax.broadcasted_iota(jnp.int32, (_ROWS, 1), 0)

    def block_body(b, _):
        oh = (row_iota == b).astype(f32)
        bx1 = _dotf(x1, oh, ((0,), (0,)))
        by1 = _dotf(y1, oh, ((0,), (0,)))
        bx2 = _dotf(x2, oh, ((0,), (0,)))
        by2 = _dotf(y2, oh, ((0,), (0,)))
        ba = _dotf(areas, oh, ((0,), (0,)))
        rx1 = x1s[pl.ds(b, 1), :]
        ry1 = y1s[pl.ds(b, 1), :]
        rx2 = x2s[pl.ds(b, 1), :]
        ry2 = y2s[pl.ds(b, 1), :]
        ra = ars[pl.ds(b, 1), :]

        s_bb = _iou_gt(bx1, by1, bx2, by2, ba,
                       rx1, ry1, rx2, ry2, ra).astype(f32) * tri
        init = keep_ref[pl.ds(b, 1), :]

        def fcond(c):
            prev, cur = c
            return jnp.any(prev != cur)

        def fbody(c):
            _, cur = c
            cnt = _dotf(cur, s_bb, ((1,), (0,)))
            return (cur, init * (cnt == 0.0).astype(f32))

        _, kb = jax.lax.while_loop(
            fcond, fbody, (-jnp.ones((1, _LANES), f32), init))
        keep_ref[pl.ds(b, 1), :] = kb
        kb_any = jnp.sum(kb) > 0.0

        def row_body(r, _):
            alive = keep_ref[pl.ds(r, 1), :]

            @pl.when((r > b) & kb_any & (jnp.sum(alive) > 0.0))
            def _():
                tx1 = x1s[pl.ds(r, 1), :]
                ty1 = y1s[pl.ds(r, 1), :]
                tx2 = x2s[pl.ds(r, 1), :]
                ty2 = y2s[pl.ds(r, 1), :]
                ta = ars[pl.ds(r, 1), :]
                s_br = _iou_gt(bx1, by1, bx2, by2, ba,
                               tx1, ty1, tx2, ty2, ta).astype(f32)
                cnt = _dotf(kb, s_br, ((1,), (0,)))
                keep_ref[pl.ds(r, 1), :] = alive * (cnt == 0.0).astype(f32)
            return 0

        jax.lax.fori_loop(0, _ROWS, row_body, 0)
        return 0

    jax.lax.fori_loop(0, _ROWS, block_body, 0)

    keep = keep_ref[...]
    fscore = jnp.where(keep > 0.0, sc, -1.0)

    inc128 = (li <= lj).astype(f32)
    r96i = jax.lax.broadcasted_iota(jnp.int32, (_ROWS, _ROWS), 1)
    r96j = jax.lax.broadcasted_iota(jnp.int32, (_ROWS, _ROWS), 0)
    strict96 = (r96i < r96j).astype(f32)
    notk = 1.0 - keep
    prefk = _dotf(keep, inc128, ((1,), (0,))) + _dotf(
        strict96, jnp.sum(keep, axis=1, keepdims=True), ((1,), (0,)))
    prefn = _dotf(notk, inc128, ((1,), (0,))) + _dotf(
        strict96, jnp.sum(notk, axis=1, keepdims=True), ((1,), (0,)))
    total_kept = jnp.sum(keep)
    rank = jnp.where(keep > 0.0, prefk - 1.0, total_kept + prefn - 1.0)

    rank_s[...] = rank
    fsc_s[...] = fscore
    p_col = jax.lax.broadcasted_iota(jnp.int32, (2048, 1), 0).astype(f32)
    out_ref[...] = jnp.zeros((5, 2048), f32)

    def gather_body(c, _):
        r0 = c * 8
        rank_flat = rank_s[pl.ds(r0, 8), :].reshape(1, 8 * _LANES)
        onehot_t = (p_col == rank_flat).astype(f32)
        data = jnp.concatenate([
            x1s[pl.ds(r0, 8), :].reshape(1, 8 * _LANES),
            y1s[pl.ds(r0, 8), :].reshape(1, 8 * _LANES),
            x2s[pl.ds(r0, 8), :].reshape(1, 8 * _LANES),
            y2s[pl.ds(r0, 8), :].reshape(1, 8 * _LANES),
            fsc_s[pl.ds(r0, 8), :].reshape(1, 8 * _LANES),
        ], axis=0)
        out_ref[...] = out_ref[...] + _dotf(data, onehot_t, ((1,), (1,)))
        return 0

    jax.lax.fori_loop(0, _ROWS // 8, gather_body, 0)


def _nms_sort(top_scores, props_raw):
    f32 = jnp.float32
    npad = _NPAD - top_scores.shape[0]
    spad = jnp.concatenate([top_scores, jnp.zeros((npad,), f32)])
    cpad = jnp.concatenate([props_raw, jnp.zeros((npad, 4), f32)], axis=0)
    coords = cpad.T.reshape(4, _ROWS, _LANES)
    out = pl.pallas_call(
        _nms_kernel,
        out_shape=jax.ShapeDtypeStruct((5, 2048), f32),
        scratch_shapes=[pltpu.VMEM((_ROWS, _LANES), f32)] * 8,
    )(spad.reshape(_ROWS, _LANES), coords[0], coords[1], coords[2], coords[3])
    final_boxes = out[:4, :_RPN_TOPK].T
    final_scores = out[4, :_RPN_TOPK]
    return final_boxes, final_scores


def kernel(image, feat, W1, b1, Wc, bc, Wr, br):
    N, _, Hf, Wf = feat.shape
    H_img, W_img = image.shape[-2], image.shape[-1]
    cls_mat, reg_mat = _conv_backbone(feat, W1, b1, Wc, bc, Wr, br)
    cls_scores = cls_mat.T.reshape(-1)
    reg_flat = reg_mat.reshape(_NUM_ANCHORS, 4, 4096).transpose(2, 0, 1).reshape(-1, 4)
    anchors = _gen_anchors(H_img, W_img, Hf, Wf, feat.dtype)
    proposals = _decode(anchors, lax.stop_gradient(reg_flat))
    scores = jax.nn.sigmoid(cls_scores)
    k = min(_PRENMS_TOPK, scores.shape[0])
    top_scores, top_idx = lax.top_k(scores, k)
    props = proposals[top_idx]
    final_boxes, final_scores = _nms_sort(top_scores, props)
    return (final_boxes, final_scores)
